# Initial kernel scaffold; baseline (speedup 1.0000x reference)
import jax
import jax.numpy as jnp
from jax.experimental import pallas as pl


def kernel(verts, edges, faces, normals, fc2_w, fc2_b, w1, r1, b1, w2, r2, b2, w3, r3, b3, w4, r4, b4, w5, r5, b5, w6, r6, b6):
    raise NotImplementedError("write your pallas kernel here")


# speedup vs baseline: 1.2209x; 1.2209x over previous
import functools

import jax
import jax.numpy as jnp
from jax import lax
from jax.experimental import pallas as pl
from jax.experimental.pallas import tpu as pltpu
from jax.experimental.pallas import tpu_sc as plsc

N = 10000
E = 160000
F = 20000
K = 20

NC, NS, L = 2, 16, 16
NW = NC * NS
CHUNK = 128
ROWS_PER_SUB = N // NS

_MESH = plsc.VectorSubcoreMesh(core_axis_name="c", subcore_axis_name="s")
_SC_PARAMS = pltpu.CompilerParams(use_tc_tiling_on_sc=False)


def _pad_rows(a, total):
    return jnp.concatenate(
        [a, jnp.zeros((total - a.shape[0],) + a.shape[1:], a.dtype)], axis=0)


def _sc_gather(table, idx):
    Ltot = idx.shape[0]
    D = table.shape[1]
    per_w = Ltot // NW
    n_chunks = per_w // CHUNK

    def body(table_ref, idx_ref, out_ref, idx_v, rows_v, sem):
        wid = lax.axis_index("s") * NC + lax.axis_index("c")

        def step(t, _):
            base = wid * per_w + t * CHUNK
            pltpu.sync_copy(idx_ref.at[pl.ds(base, CHUNK)], idx_v)
            pltpu.async_copy(table_ref.at[idx_v], rows_v, sem).wait()
            pltpu.sync_copy(rows_v, out_ref.at[pl.ds(base, CHUNK)])
            return _

        lax.fori_loop(0, n_chunks, step, 0)

    return pl.kernel(
        body,
        mesh=_MESH,
        compiler_params=_SC_PARAMS,
        out_type=jax.ShapeDtypeStruct((Ltot, D), jnp.float32),
        scratch_types=[
            pltpu.VMEM((CHUNK,), jnp.int32),
            pltpu.VMEM((CHUNK, D), jnp.float32),
            pltpu.SemaphoreType.DMA,
        ],
    )(table, idx)


def _sc_scatter_add(idx, val, n_rows, d):
    Ltot = idx.shape[0]
    per_w = Ltot // NW
    n_chunks = per_w // CHUNK
    rps = n_rows // NS

    def body(idx_ref, val_ref, zeros_ref, out_ref, idx_v, val_v, acc):
        cid = lax.axis_index("c")
        sid = lax.axis_index("s")
        wid = sid * NC + cid
        pltpu.sync_copy(zeros_ref, acc.at[pl.ds(sid * rps, rps)])
        plsc.subcore_barrier()

        def step(t, _):
            base = wid * per_w + t * CHUNK
            pltpu.sync_copy(idx_ref.at[pl.ds(base, CHUNK)], idx_v)
            pltpu.sync_copy(val_ref.at[pl.ds(base, CHUNK)], val_v)
            pltpu.sync_copy(val_v, acc.at[idx_v], add=True)
            return _

        lax.fori_loop(0, n_chunks, step, 0)
        plsc.subcore_barrier()
        pltpu.sync_copy(acc.at[pl.ds(sid * rps, rps)],
                        out_ref.at[cid, pl.ds(sid * rps, rps)])

    zeros = jnp.zeros((rps, d), jnp.float32)
    return pl.kernel(
        body,
        mesh=_MESH,
        compiler_params=_SC_PARAMS,
        out_type=jax.ShapeDtypeStruct((NC, n_rows, d), jnp.float32),
        scratch_types=[
            pltpu.VMEM((CHUNK,), jnp.int32),
            pltpu.VMEM((CHUNK, d), jnp.float32),
            pltpu.VMEM_SHARED((n_rows, d), jnp.float32),
        ],
    )(idx, val, zeros)


def _sc_edge_pass(table, r0, r1, c0, c1, dst, fop):
    Ep = r0.shape[0]
    per_w = Ep // NW
    n_chunks = per_w // CHUNK
    rps = ROWS_PER_SUB
    nj = fop // L

    def body(table_ref, r0_ref, r1_ref, c0_ref, c1_ref, dst_ref, zeros_ref,
             out_ref, i0_v, i1_v, d_v, a0_v, a1_v, rows0, rows1, acc,
             sem0, sem1):
        cid = lax.axis_index("c")
        sid = lax.axis_index("s")
        wid = sid * NC + cid
        pltpu.sync_copy(zeros_ref, acc.at[pl.ds(sid * rps, rps)])
        plsc.subcore_barrier()

        def step(t, _):
            base = wid * per_w + t * CHUNK
            pltpu.sync_copy(r0_ref.at[pl.ds(base, CHUNK)], i0_v)
            pltpu.sync_copy(r1_ref.at[pl.ds(base, CHUNK)], i1_v)
            pltpu.sync_copy(c0_ref.at[pl.ds(base, CHUNK)], a0_v)
            pltpu.sync_copy(c1_ref.at[pl.ds(base, CHUNK)], a1_v)
            pltpu.sync_copy(dst_ref.at[pl.ds(base, CHUNK)], d_v)
            g0 = pltpu.async_copy(table_ref.at[i0_v], rows0, sem0)
            g1 = pltpu.async_copy(table_ref.at[i1_v], rows1, sem1)
            g0.wait()
            g1.wait()

            def interp(e, _):
                a0 = a0_v[e, :]
                a1 = a1_v[e, :]
                for j in range(nj):
                    sl = pl.ds(j * L, L)
                    rows0[e, sl] = a0 * rows0[e, sl] + a1 * rows1[e, sl]
                return _

            lax.fori_loop(0, CHUNK, interp, 0)
            pltpu.sync_copy(rows0, acc.at[d_v], add=True)
            return _

        lax.fori_loop(0, n_chunks, step, 0)
        plsc.subcore_barrier()
        pltpu.sync_copy(acc.at[pl.ds(sid * rps, rps)],
                        out_ref.at[cid, pl.ds(sid * rps, rps)])

    zeros = jnp.zeros((rps, fop), jnp.float32)
    return pl.kernel(
        body,
        mesh=_MESH,
        compiler_params=_SC_PARAMS,
        out_type=jax.ShapeDtypeStruct((NC, N, fop), jnp.float32),
        scratch_types=[
            pltpu.VMEM((CHUNK,), jnp.int32),
            pltpu.VMEM((CHUNK,), jnp.int32),
            pltpu.VMEM((CHUNK,), jnp.int32),
            pltpu.VMEM((CHUNK, L), jnp.float32),
            pltpu.VMEM((CHUNK, L), jnp.float32),
            pltpu.VMEM((CHUNK, fop), jnp.float32),
            pltpu.VMEM((CHUNK, fop), jnp.float32),
            pltpu.VMEM_SHARED((N, fop), jnp.float32),
            pltpu.SemaphoreType.DMA,
            pltpu.SemaphoreType.DMA,
        ],
    )(table, r0, r1, c0, c1, dst, zeros)


def _tc_prep_edges(va, vb, src):
    def body(va_ref, vb_ref, src_ref, r0_ref, r1_ref, c0_ref, c1_ref):
        dx = va_ref[:, 0:1] - vb_ref[:, 0:1]
        dy = va_ref[:, 1:2] - vb_ref[:, 1:2]
        dz = va_ref[:, 2:3] - vb_ref[:, 2:3]
        u = jnp.minimum(jnp.sqrt(dx * dx + dy * dy + dz * dz), 1.0) * (K - 1.0)
        i0 = jnp.clip(jnp.floor(u).astype(jnp.int32), 0, K - 2)
        frac = u - i0.astype(jnp.float32)
        r0 = src_ref[...] * K + i0
        r0_ref[...] = r0
        r1_ref[...] = r0 + 1
        c0_ref[...] = jnp.broadcast_to(1.0 - frac, c0_ref.shape)
        c1_ref[...] = jnp.broadcast_to(frac, c1_ref.shape)

    nb = 40
    b = E // nb
    return pl.pallas_call(
        body,
        grid=(nb,),
        in_specs=[pl.BlockSpec((b, 8), lambda i: (i, 0)),
                  pl.BlockSpec((b, 8), lambda i: (i, 0)),
                  pl.BlockSpec((b, 1), lambda i: (i, 0))],
        out_specs=[pl.BlockSpec((b, 1), lambda i: (i, 0)),
                   pl.BlockSpec((b, 1), lambda i: (i, 0)),
                   pl.BlockSpec((b, L), lambda i: (i, 0)),
                   pl.BlockSpec((b, L), lambda i: (i, 0))],
        out_shape=[jax.ShapeDtypeStruct((E, 1), jnp.int32),
                   jax.ShapeDtypeStruct((E, 1), jnp.int32),
                   jax.ShapeDtypeStruct((E, L), jnp.float32),
                   jax.ShapeDtypeStruct((E, L), jnp.float32)],
    )(va, vb, src)


def _tc_prep_faces(fv0, fv1, fv2):
    def body(a_ref, b_ref, c_ref, area_ref):
        t1 = [b_ref[:, i:i + 1] - a_ref[:, i:i + 1] for i in range(3)]
        t2 = [c_ref[:, i:i + 1] - a_ref[:, i:i + 1] for i in range(3)]
        cx = t1[1] * t2[2] - t1[2] * t2[1]
        cy = t1[2] * t2[0] - t1[0] * t2[2]
        cz = t1[0] * t2[1] - t1[1] * t2[0]
        area = 0.5 * jnp.sqrt(cx * cx + cy * cy + cz * cz)
        area_ref[...] = jnp.broadcast_to(area, area_ref.shape)

    nb = 10
    b = F // nb
    return pl.pallas_call(
        body,
        grid=(nb,),
        in_specs=[pl.BlockSpec((b, 8), lambda i: (i, 0))] * 3,
        out_specs=pl.BlockSpec((b, 8), lambda i: (i, 0)),
        out_shape=jax.ShapeDtypeStruct((F, 8), jnp.float32),
    )(fv0, fv1, fv2)


def _tc_x0(verts, normals, area_parts, cnt_parts, fc2_wt, fc2_b):
    def body(v_ref, n_ref, ap_ref, cp_ref, w_ref, b_ref, x_ref, ci_ref):
        ta = ap_ref[0, :, 0:1] + ap_ref[1, :, 0:1]
        cnt = cp_ref[0, :, 0:1] + cp_ref[1, :, 0:1]
        ci_ref[...] = 1.0 / jnp.maximum(cnt, 1.0)
        feat = jnp.concatenate([v_ref[...], ta, n_ref[...]], axis=1)
        x_ref[...] = jnp.dot(feat, w_ref[...],
                             preferred_element_type=jnp.float32) + b_ref[...]

    return pl.pallas_call(
        body,
        out_shape=[jax.ShapeDtypeStruct((N, 16), jnp.float32),
                   jax.ShapeDtypeStruct((N, 1), jnp.float32)],
    )(verts, normals, area_parts, cnt_parts, fc2_wt, fc2_b)


def _tc_matmul(x, w):
    fin = x.shape[1]
    cols = w.shape[1]
    cb = 640 if cols % 640 == 0 else cols
    nr = 5
    rb = N // nr

    def body(x_ref, w_ref, o_ref):
        o_ref[...] = jnp.dot(x_ref[...], w_ref[...],
                             preferred_element_type=jnp.float32)

    return pl.pallas_call(
        body,
        grid=(nr, cols // cb),
        in_specs=[pl.BlockSpec((rb, fin), lambda i, j: (i, 0)),
                  pl.BlockSpec((fin, cb), lambda i, j: (0, j))],
        out_specs=pl.BlockSpec((rb, cb), lambda i, j: (i, j)),
        out_shape=jax.ShapeDtypeStruct((N, cols), jnp.float32),
    )(x, w)


def _tc_epilogue(parts, x, root, bias, cnt_inv, fo, act):
    fop = parts.shape[2]
    fin = x.shape[1]
    nr = 5
    rb = N // nr

    def body(p_ref, x_ref, r_ref, b_ref, ci_ref, y_ref):
        agg = (p_ref[0] + p_ref[1])[:, :fo] * ci_ref[...]
        y = agg + jnp.dot(x_ref[...], r_ref[...],
                          preferred_element_type=jnp.float32) + b_ref[...]
        if act:
            y = jnp.where(y > 0, y, jnp.exp(jnp.minimum(y, 0.0)) - 1.0)
        y_ref[...] = y

    return pl.pallas_call(
        body,
        grid=(nr,),
        in_specs=[pl.BlockSpec((2, rb, fop), lambda i: (0, i, 0)),
                  pl.BlockSpec((rb, fin), lambda i: (i, 0)),
                  pl.BlockSpec((fin, fo), lambda i: (0, 0)),
                  pl.BlockSpec((1, fo), lambda i: (0, 0)),
                  pl.BlockSpec((rb, 1), lambda i: (i, 0))],
        out_specs=pl.BlockSpec((rb, fo), lambda i: (i, 0)),
        out_shape=jax.ShapeDtypeStruct((N, fo), jnp.float32),
    )(parts, x, root, bias, cnt_inv)


FOS = [64, 128, 64, 32, 16, 1]
FOPS = [64, 128, 64, 32, 16, 16]


def kernel(verts, edges, faces, normals, fc2_w, fc2_b,
           w1, r1, b1, w2, r2, b2, w3, r3, b3,
           w4, r4, b4, w5, r5, b5, w6, r6, b6):
    ws = [w1, w2, w3, w4, w5, w6]
    rs = [r1, r2, r3, r4, r5, r6]
    bs = [b1, b2, b3, b4, b5, b6]
    src, dst = edges[0], edges[1]
    f0, f1c, f2c = faces[:, 0], faces[:, 1], faces[:, 2]

    verts8 = jnp.concatenate([verts, jnp.zeros((N, 5), jnp.float32)], axis=1)

    gidx = jnp.concatenate([src, dst, f0, f1c, f2c])
    gidx = jnp.concatenate(
        [gidx, jnp.zeros((380928 - gidx.shape[0],), jnp.int32)])
    rows = _sc_gather(verts8, gidx)
    va, vb = rows[:E], rows[E:2 * E]
    fv0, fv1, fv2 = (rows[2 * E:2 * E + F], rows[2 * E + F:2 * E + 2 * F],
                     rows[2 * E + 2 * F:2 * E + 3 * F])

    er0, er1, ec0, ec1 = _tc_prep_edges(va, vb, src[:, None])
    area8 = _tc_prep_faces(fv0, fv1, fv2)

    Ep = 163840
    dst_p = _pad_rows(dst, Ep)
    ones8 = _pad_rows(jnp.ones((E, 8), jnp.float32), Ep)
    cnt_parts = _sc_scatter_add(dst_p, ones8, N, 8)
    fidx = _pad_rows(jnp.concatenate([f0, f1c, f2c]), 61440)
    fval = _pad_rows(jnp.concatenate([area8, area8, area8]), 61440)
    area_parts = _sc_scatter_add(fidx, fval, N, 8)

    x, cnt_inv = _tc_x0(verts, normals, area_parts, cnt_parts,
                        fc2_w.T, fc2_b[None, :])

    r0p = _pad_rows(er0[:, 0], Ep)
    r1p = jnp.concatenate([er1[:, 0], jnp.ones((Ep - E,), jnp.int32)])
    c0p = _pad_rows(ec0, Ep)
    c1p = _pad_rows(ec1, Ep)

    for i in range(6):
        fo, fop = FOS[i], FOPS[i]
        w = ws[i]
        if fop != fo:
            w = jnp.concatenate(
                [w, jnp.zeros((K, w.shape[1], fop - fo), jnp.float32)], axis=2)
        wflat = jnp.transpose(w, (1, 0, 2)).reshape(w.shape[1], K * fop)
        table = _tc_matmul(x, wflat).reshape(N * K, fop)
        parts = _sc_edge_pass(table, r0p, r1p, c0p, c1p, dst_p, fop)
        x = _tc_epilogue(parts, x, rs[i], bs[i][None, :], cnt_inv,
                         fo, act=(i < 5))
    return x[:, 0]
